# baseline (speedup 1.0000x reference)
import functools

import jax
import jax.numpy as jnp
from jax import lax
from jax.experimental import pallas as pl
from jax.experimental.pallas import tpu as pltpu
from jax.experimental.pallas import tpu_sc as plsc

B = 16384
D = 64
V = 1000000
EPS_BN = 1e-5
EPS_NORM = 1e-12

NC, NS = 2, 16
NW = NC * NS
NCOLS = (V + 127) // 128
CPW = (NCOLS + NW - 1) // NW
NRING = 8


def _sc_gather_body(table_t, last_t, idx_hbm, out_flat,
                    ids_v, mid_v, mpos_v, col_v, col64_v, row_v, sem_r, sem_c):
    wid = lax.axis_index("s") * NC + lax.axis_index("c")
    c0 = wid * CPW
    nc = jnp.minimum(CPW, NCOLS - 1 - c0)
    nc = jnp.maximum(nc, 0)
    pltpu.sync_copy(idx_hbm, ids_v)

    iota = lax.iota(jnp.int32, 16)

    def compress(k, off):
        v = ids_v[pl.ds(k * 16, 16)]
        cm = v >> 7
        m = (cm >= c0) & (cm < c0 + CPW)
        plsc.store_compressed(mid_v.at[pl.ds(off, 16)], v, mask=m)
        plsc.store_compressed(mpos_v.at[pl.ds(off, 16)], iota + k * 16, mask=m)
        return off + plsc.all_reduce_population_count(m)[0]

    nmatch = lax.fori_loop(0, B // 16, compress, jnp.int32(0))
    mid_v[pl.ds(nmatch, 16)] = jnp.full((16,), -128, jnp.int32)
    nm16 = (nmatch + 15) >> 4

    def extract_col(c, colbuf, dmas):
        def scan(k, dmas):
            v = mid_v[pl.ds(k * 16, 16)]
            m = (v >> 7) == (c0 + c)
            cnt = plsc.all_reduce_population_count(m)[0]

            def hit(dmas):
                pv = mpos_v[pl.ds(k * 16, 16)]
                mi = jnp.where(m, 1, 0)

                def do_lane(j, dmas):
                    @pl.when(dmas >= NRING)
                    def _():
                        pltpu.make_async_copy(
                            row_v.at[pl.ds(0, D)],
                            out_flat.at[pl.ds(0, D)], sem_r).wait()

                    idj = v[j]
                    pos = pv[j]
                    lane = idj & 127
                    slot = (dmas % NRING) * D

                    def gather_feats(g, _):
                        vals = plsc.load_gather(
                            colbuf,
                            [iota + g * 16, jnp.full((16,), lane, jnp.int32)])
                        row_v[pl.ds(slot + g * 16, 16)] = vals
                        return 0

                    lax.fori_loop(0, D // 16, gather_feats, 0)
                    pltpu.async_copy(row_v.at[pl.ds(slot, D)],
                                     out_flat.at[pl.ds(pos * D, D)], sem_r)
                    return dmas + 1

                for j in range(16):
                    dmas = lax.cond(mi[j] != 0, lambda d, j=j: do_lane(j, d),
                                    lambda d: d, dmas)
                return dmas

            return lax.cond(cnt > 0, hit, lambda d: d, dmas)

        return lax.fori_loop(0, nm16, scan, dmas)

    def fetch(c, buf_slot):
        return pltpu.async_copy(
            table_t.at[:, pl.ds((c0 + c) * 128, 128)],
            col_v.at[buf_slot], sem_c)

    @pl.when(nc > 0)
    def _():
        fetch(0, 0)

    def col_loop(c, dmas):
        pltpu.make_async_copy(table_t.at[:, pl.ds(0, 128)],
                              col_v.at[c % 2], sem_c).wait()

        @pl.when(c + 1 < nc)
        def _():
            fetch(c + 1, (c + 1) % 2)

        return extract_col(c, col_v.at[c % 2], dmas)

    dmas = lax.fori_loop(0, nc, col_loop, jnp.int32(0))

    @pl.when(c0 + CPW >= NCOLS)
    def _():
        pltpu.sync_copy(last_t, col64_v)

    dmas = lax.cond(c0 + CPW >= NCOLS,
                    lambda d: extract_col(NCOLS - 1 - c0, col64_v, d),
                    lambda d: d, dmas)

    def drain(i, _):
        pltpu.make_async_copy(row_v.at[pl.ds(0, D)],
                              out_flat.at[pl.ds(0, D)], sem_r).wait()
        return 0

    lax.fori_loop(0, jnp.minimum(dmas, NRING), drain, 0)


@functools.cache
def _sc_gather():
    return pl.kernel(
        _sc_gather_body,
        mesh=plsc.VectorSubcoreMesh(core_axis_name="c", subcore_axis_name="s"),
        compiler_params=pltpu.CompilerParams(use_tc_tiling_on_sc=True,
                                             needs_layout_passes=False),
        out_type=jax.ShapeDtypeStruct((B * D,), jnp.float32),
        scratch_types=[pltpu.VMEM((B,), jnp.int32),
                       pltpu.VMEM((B + 16, ), jnp.int32),
                       pltpu.VMEM((B + 16, ), jnp.int32),
                       pltpu.VMEM((2, D, 128), jnp.float32),
                       pltpu.VMEM((D, D), jnp.float32),
                       pltpu.VMEM((NRING * D,), jnp.float32),
                       pltpu.SemaphoreType.DMA,
                       pltpu.SemaphoreType.DMA],
    )


def _bn_relu(y, g, c):
    m = jnp.mean(y, axis=0, keepdims=True)
    d = y - m
    v = jnp.mean(d * d, axis=0, keepdims=True)
    return jnp.maximum(g * d / jnp.sqrt(v + EPS_BN) + c, 0.0)


def _tower(x, W1, b1, g1, c1, W2, b2, g2, c2, W3, b3):
    y = jnp.dot(x, W1[...], preferred_element_type=jnp.float32) + b1[...]
    y = _bn_relu(y, g1[...], c1[...])
    y = jnp.dot(y, W2[...], preferred_element_type=jnp.float32) + b2[...]
    y = _bn_relu(y, g2[...], c2[...])
    return jnp.dot(y, W3[...], preferred_element_type=jnp.float32) + b3[...]


def _l2norm(x):
    n = jnp.sqrt(jnp.sum(x * x, axis=-1, keepdims=True))
    return x / jnp.maximum(n, EPS_NORM)


def _user_tower_body(ue, W1, b1, g1, c1, W2, b2, g2, c2, W3, b3, out_ref):
    out_ref[...] = _l2norm(_tower(ue[...], W1, b1, g1, c1, W2, b2, g2, c2, W3, b3))


def _item_tower_body(ie, W1, b1, g1, c1, W2, b2, g2, c2, W3, b3, un, out_ref):
    i = _l2norm(_tower(ie[...], W1, b1, g1, c1, W2, b2, g2, c2, W3, b3))
    out_ref[...] = jnp.sum(i * un[...], axis=-1, keepdims=True)


_user_tower = pl.pallas_call(
    _user_tower_body,
    out_shape=jax.ShapeDtypeStruct((B, 32), jnp.float32),
    compiler_params=pltpu.CompilerParams(vmem_limit_bytes=48 * 1024 * 1024),
)

_item_tower = pl.pallas_call(
    _item_tower_body,
    out_shape=jax.ShapeDtypeStruct((B, 1), jnp.float32),
    compiler_params=pltpu.CompilerParams(vmem_limit_bytes=48 * 1024 * 1024),
)


def _tower_args(tp):
    W, b = tp["W"], tp["b"]
    g, c = tp["gamma"], tp["beta"]
    r = lambda v: v.reshape(1, -1)
    return (W[0], r(b[0]), r(g[0]), r(c[0]),
            W[1], r(b[1]), r(g[1]), r(c[1]),
            W[2], r(b[2]))


def kernel(user_ids, item_ids, params):
    gather = _sc_gather()
    ut, it = params["user_table"], params["item_table"]
    uef = gather(ut.T, ut[(NCOLS - 1) * 128:].T, user_ids.astype(jnp.int32))
    ief = gather(it.T, it[(NCOLS - 1) * 128:].T, item_ids.astype(jnp.int32))
    ue = uef.reshape(B, D)
    ie = ief.reshape(B, D)
    un = _user_tower(ue, *_tower_args(params["user_tower"]))
    scores = _item_tower(ie, *_tower_args(params["item_tower"]), un)
    return scores.reshape(B)


# speedup vs baseline: 1.3532x; 1.3532x over previous
import functools

import jax
import jax.numpy as jnp
from jax import lax
from jax.experimental import pallas as pl
from jax.experimental.pallas import tpu as pltpu
from jax.experimental.pallas import tpu_sc as plsc

B = 16384
D = 64
V = 1000000
EPS_BN = 1e-5
EPS_NORM = 1e-12

NC, NS = 2, 16
NW = NC * NS
NCOLS = (V + 127) // 128
CPW = 248
GW = 4
NRING = 16


def _sc_gather_body(table_t, last_t, idx_hbm, out_flat,
                    ids_v, mid_v, mpos_v, col_v, col64_v, row_v, sem_r, sem_c):
    wid = lax.axis_index("s") * NC + lax.axis_index("c")
    c0 = wid * CPW
    ng = jnp.clip(NCOLS - 1 - c0, 0, CPW) // GW
    pltpu.sync_copy(idx_hbm, ids_v)

    iota = lax.iota(jnp.int32, 16)

    def compress(k, off):
        v = ids_v[pl.ds(k * 16, 16)]
        cm = v >> 7
        m = (cm >= c0) & (cm < c0 + CPW)
        plsc.store_compressed(mid_v.at[pl.ds(off, 16)], v, mask=m)
        plsc.store_compressed(mpos_v.at[pl.ds(off, 16)], iota + k * 16, mask=m)
        return off + plsc.all_reduce_population_count(m)[0]

    nmatch = lax.fori_loop(0, B // 16, compress, jnp.int32(0))
    mid_v[pl.ds(nmatch, 16)] = jnp.full((16,), -128, jnp.int32)
    nm16 = (nmatch + 15) >> 4

    def extract_grp(shift, mval, lmask, colbuf, dmas):
        def scan(k, dmas):
            v = mid_v[pl.ds(k * 16, 16)]
            m = (v >> shift) == mval
            cnt = plsc.all_reduce_population_count(m)[0]

            def hit(dmas):
                pv = mpos_v[pl.ds(k * 16, 16)]
                mi = jnp.where(m, 1, 0)

                def do_lane(j, dmas):
                    @pl.when(dmas >= NRING)
                    def _():
                        pltpu.make_async_copy(
                            row_v.at[pl.ds(0, D)],
                            out_flat.at[pl.ds(0, D)], sem_r).wait()

                    idj = v[j]
                    pos = pv[j]
                    lane = idj & lmask
                    slot = (dmas % NRING) * D

                    def gather_feats(g, _):
                        vals = plsc.load_gather(
                            colbuf,
                            [iota + g * 16, jnp.full((16,), lane, jnp.int32)])
                        row_v[pl.ds(slot + g * 16, 16)] = vals
                        return 0

                    lax.fori_loop(0, D // 16, gather_feats, 0)
                    pltpu.async_copy(row_v.at[pl.ds(slot, D)],
                                     out_flat.at[pl.ds(pos * D, D)], sem_r)
                    return dmas + 1

                for j in range(16):
                    dmas = lax.cond(mi[j] != 0, lambda d, j=j: do_lane(j, d),
                                    lambda d: d, dmas)
                return dmas

            return lax.cond(cnt > 0, hit, lambda d: d, dmas)

        return lax.fori_loop(0, nm16, scan, dmas)

    def fetch(g, buf_slot):
        return pltpu.async_copy(
            table_t.at[:, pl.ds((c0 + g * GW) * 128, GW * 128)],
            col_v.at[buf_slot], sem_c)

    @pl.when(ng > 0)
    def _():
        fetch(0, 0)

    def grp_loop(g, dmas):
        pltpu.make_async_copy(table_t.at[:, pl.ds(0, GW * 128)],
                              col_v.at[g % 2], sem_c).wait()

        @pl.when(g + 1 < ng)
        def _():
            fetch(g + 1, (g + 1) % 2)

        return extract_grp(9, c0 // GW + g, GW * 128 - 1,
                           col_v.at[g % 2], dmas)

    dmas = lax.fori_loop(0, ng, grp_loop, jnp.int32(0))

    @pl.when(c0 + CPW >= NCOLS)
    def _():
        pltpu.sync_copy(last_t, col64_v)

    dmas = lax.cond(c0 + CPW >= NCOLS,
                    lambda d: extract_grp(7, NCOLS - 1, 127, col64_v, d),
                    lambda d: d, dmas)

    def drain(i, _):
        pltpu.make_async_copy(row_v.at[pl.ds(0, D)],
                              out_flat.at[pl.ds(0, D)], sem_r).wait()
        return 0

    lax.fori_loop(0, jnp.minimum(dmas, NRING), drain, 0)


@functools.cache
def _sc_gather():
    return pl.kernel(
        _sc_gather_body,
        mesh=plsc.VectorSubcoreMesh(core_axis_name="c", subcore_axis_name="s"),
        compiler_params=pltpu.CompilerParams(use_tc_tiling_on_sc=True,
                                             needs_layout_passes=False),
        out_type=jax.ShapeDtypeStruct((B * D,), jnp.float32),
        scratch_types=[pltpu.VMEM((B,), jnp.int32),
                       pltpu.VMEM((B + 16, ), jnp.int32),
                       pltpu.VMEM((B + 16, ), jnp.int32),
                       pltpu.VMEM((2, D, GW * 128), jnp.float32),
                       pltpu.VMEM((D, D), jnp.float32),
                       pltpu.VMEM((NRING * D,), jnp.float32),
                       pltpu.SemaphoreType.DMA,
                       pltpu.SemaphoreType.DMA],
    )


def _bn_relu(y, g, c):
    m = jnp.mean(y, axis=0, keepdims=True)
    d = y - m
    v = jnp.mean(d * d, axis=0, keepdims=True)
    return jnp.maximum(g * d / jnp.sqrt(v + EPS_BN) + c, 0.0)


def _tower(x, W1, b1, g1, c1, W2, b2, g2, c2, W3, b3):
    y = jnp.dot(x, W1[...], preferred_element_type=jnp.float32) + b1[...]
    y = _bn_relu(y, g1[...], c1[...])
    y = jnp.dot(y, W2[...], preferred_element_type=jnp.float32) + b2[...]
    y = _bn_relu(y, g2[...], c2[...])
    return jnp.dot(y, W3[...], preferred_element_type=jnp.float32) + b3[...]


def _l2norm(x):
    n = jnp.sqrt(jnp.sum(x * x, axis=-1, keepdims=True))
    return x / jnp.maximum(n, EPS_NORM)


def _user_tower_body(ue, W1, b1, g1, c1, W2, b2, g2, c2, W3, b3, out_ref):
    out_ref[...] = _l2norm(_tower(ue[...], W1, b1, g1, c1, W2, b2, g2, c2, W3, b3))


def _item_tower_body(ie, W1, b1, g1, c1, W2, b2, g2, c2, W3, b3, un, out_ref):
    i = _l2norm(_tower(ie[...], W1, b1, g1, c1, W2, b2, g2, c2, W3, b3))
    out_ref[...] = jnp.sum(i * un[...], axis=-1, keepdims=True)


_user_tower = pl.pallas_call(
    _user_tower_body,
    out_shape=jax.ShapeDtypeStruct((B, 32), jnp.float32),
    compiler_params=pltpu.CompilerParams(vmem_limit_bytes=48 * 1024 * 1024),
)

_item_tower = pl.pallas_call(
    _item_tower_body,
    out_shape=jax.ShapeDtypeStruct((B, 1), jnp.float32),
    compiler_params=pltpu.CompilerParams(vmem_limit_bytes=48 * 1024 * 1024),
)


def _tower_args(tp):
    W, b = tp["W"], tp["b"]
    g, c = tp["gamma"], tp["beta"]
    r = lambda v: v.reshape(1, -1)
    return (W[0], r(b[0]), r(g[0]), r(c[0]),
            W[1], r(b[1]), r(g[1]), r(c[1]),
            W[2], r(b[2]))


def kernel(user_ids, item_ids, params):
    gather = _sc_gather()
    ut, it = params["user_table"], params["item_table"]
    uef = gather(ut.T, ut[(NCOLS - 1) * 128:].T, user_ids.astype(jnp.int32))
    ief = gather(it.T, it[(NCOLS - 1) * 128:].T, item_ids.astype(jnp.int32))
    ue = uef.reshape(B, D)
    ie = ief.reshape(B, D)
    un = _user_tower(ue, *_tower_args(params["user_tower"]))
    scores = _item_tower(ie, *_tower_args(params["item_tower"]), un)
    return scores.reshape(B)


# speedup vs baseline: 1.4054x; 1.0386x over previous
import functools

import jax
import jax.numpy as jnp
from jax import lax
from jax.experimental import pallas as pl
from jax.experimental.pallas import tpu as pltpu
from jax.experimental.pallas import tpu_sc as plsc

B = 16384
D = 64
EPS_BN = 1e-5
EPS_NORM = 1e-12

NC, NS = 2, 16
NW = NC * NS
BPW = B // NW


def _sc_gather_body(table, idx_hbm, out, idx_v, rows_v, sem):
    wid = lax.axis_index("s") * NC + lax.axis_index("c")
    base = wid * BPW
    pltpu.sync_copy(idx_hbm.at[pl.ds(base, BPW)], idx_v)

    def enqueue(k, _):
        vec = idx_v[pl.ds(k * 16, 16)]
        for j in range(16):
            pltpu.async_copy(table.at[pl.ds(vec[j], 1)],
                             rows_v.at[pl.ds(k * 16 + j, 1)], sem)
        return 0

    lax.fori_loop(0, BPW // 16, enqueue, 0)
    pltpu.make_async_copy(table.at[pl.ds(0, BPW)], rows_v, sem).wait()
    pltpu.sync_copy(rows_v, out.at[pl.ds(base, BPW)])


@functools.cache
def _sc_gather():
    return pl.kernel(
        _sc_gather_body,
        mesh=plsc.VectorSubcoreMesh(core_axis_name="c", subcore_axis_name="s"),
        compiler_params=pltpu.CompilerParams(use_tc_tiling_on_sc=True),
        out_type=jax.ShapeDtypeStruct((B, D), jnp.float32),
        scratch_types=[pltpu.VMEM((BPW,), jnp.int32),
                       pltpu.VMEM((BPW, D), jnp.float32),
                       pltpu.SemaphoreType.DMA],
    )


def _bn_relu(y, g, c):
    m = jnp.mean(y, axis=0, keepdims=True)
    d = y - m
    v = jnp.mean(d * d, axis=0, keepdims=True)
    return jnp.maximum(g * d / jnp.sqrt(v + EPS_BN) + c, 0.0)


def _tower(x, W1, b1, g1, c1, W2, b2, g2, c2, W3, b3):
    y = jnp.dot(x, W1[...], preferred_element_type=jnp.float32) + b1[...]
    y = _bn_relu(y, g1[...], c1[...])
    y = jnp.dot(y, W2[...], preferred_element_type=jnp.float32) + b2[...]
    y = _bn_relu(y, g2[...], c2[...])
    return jnp.dot(y, W3[...], preferred_element_type=jnp.float32) + b3[...]


def _l2norm(x):
    n = jnp.sqrt(jnp.sum(x * x, axis=-1, keepdims=True))
    return x / jnp.maximum(n, EPS_NORM)


def _user_tower_body(ue, W1, b1, g1, c1, W2, b2, g2, c2, W3, b3, out_ref):
    out_ref[...] = _l2norm(_tower(ue[...], W1, b1, g1, c1, W2, b2, g2, c2, W3, b3))


def _item_tower_body(ie, W1, b1, g1, c1, W2, b2, g2, c2, W3, b3, un, out_ref):
    i = _l2norm(_tower(ie[...], W1, b1, g1, c1, W2, b2, g2, c2, W3, b3))
    out_ref[...] = jnp.sum(i * un[...], axis=-1, keepdims=True)


_user_tower = pl.pallas_call(
    _user_tower_body,
    out_shape=jax.ShapeDtypeStruct((B, 32), jnp.float32),
    compiler_params=pltpu.CompilerParams(vmem_limit_bytes=48 * 1024 * 1024),
)

_item_tower = pl.pallas_call(
    _item_tower_body,
    out_shape=jax.ShapeDtypeStruct((B, 1), jnp.float32),
    compiler_params=pltpu.CompilerParams(vmem_limit_bytes=48 * 1024 * 1024),
)


def _tower_args(tp):
    W, b = tp["W"], tp["b"]
    g, c = tp["gamma"], tp["beta"]
    r = lambda v: v.reshape(1, -1)
    return (W[0], r(b[0]), r(g[0]), r(c[0]),
            W[1], r(b[1]), r(g[1]), r(c[1]),
            W[2], r(b[2]))


def kernel(user_ids, item_ids, params):
    gather = _sc_gather()
    ue = gather(params["user_table"], user_ids.astype(jnp.int32))
    ie = gather(params["item_table"], item_ids.astype(jnp.int32))
    un = _user_tower(ue, *_tower_args(params["user_tower"]))
    scores = _item_tower(ie, *_tower_args(params["item_tower"]), un)
    return scores.reshape(B)
